# baseline (speedup 1.0000x reference)
import jax
import jax.numpy as jnp
from jax.experimental import pallas as pl

_DIM = 2048
_NE = 8
_MAXIT = 3
_B = 4
_S = 4096
_SBLK = 1024
_NS = _S // _SBLK


def _norms_body(y_ref, yp_ref, an_ref, dn_ref, dr_ref):
    y = y_ref[0]
    d = y - yp_ref[0]
    a = y - an_ref[0]
    dn_ref[0] = jnp.sqrt(jnp.sum(d * d, axis=1, keepdims=True))
    dr_ref[0] = jnp.sqrt(jnp.sum(a * a, axis=1, keepdims=True))


def _route_body(dn_ref, dr_ref, it_ref, w1_ref, b1_ref, w2_ref, idx_ref, out_ref):
    delta = jnp.mean(dn_ref[...], axis=1, keepdims=True)
    drift = jnp.mean(dr_ref[...], axis=1, keepdims=True)
    delta = delta / (jnp.mean(delta) + 1e-8)
    drift = drift / (jnp.mean(drift) + 1e-8)
    clamped = jnp.minimum(idx_ref[...], _MAXIT - 1)
    sel = (jax.lax.broadcasted_iota(jnp.int32, (1, _MAXIT), 1) == clamped
           ).astype(jnp.float32)
    emb = jax.lax.dot_general(sel, it_ref[...], (((1,), (0,)), ((), ())),
                              preferred_element_type=jnp.float32)
    emb4 = jnp.broadcast_to(emb, (_B, 16))
    x = jnp.concatenate([delta, drift, emb4], axis=1)
    h = jax.lax.dot_general(x, w1_ref[...], (((1,), (1,)), ((), ())),
                            preferred_element_type=jnp.float32) + b1_ref[...]
    xh = h[:, : _NE * 4]
    gate = h[:, _NE * 4:]
    h2 = (gate * jax.lax.logistic(gate)) * xh
    logits = jax.lax.dot_general(h2, w2_ref[...], (((1,), (1,)), ((), ())),
                                 preferred_element_type=jnp.float32)
    mx = jnp.max(logits, axis=1, keepdims=True)
    iota = jax.lax.broadcasted_iota(jnp.int32, (_B, _NE), 1)
    first = jnp.min(jnp.where(logits == mx, iota, _NE), axis=1, keepdims=True)
    onehot = (iota == first).astype(jnp.float32)
    out_ref[...] = jnp.zeros((8, 128), jnp.float32)
    out_ref[0:_B, 0:_NE] = onehot


def kernel(y, y_prev, linguistic_anchor, iter_table, W1, b1, W2, iter_idx):
    dn3, dr3 = pl.pallas_call(
        _norms_body,
        grid=(_B, _NS),
        in_specs=[pl.BlockSpec((1, _SBLK, _DIM), lambda b, s: (b, s, 0))
                  for _ in range(3)],
        out_specs=[pl.BlockSpec((1, _SBLK, 1), lambda b, s: (b, s, 0))
                   for _ in range(2)],
        out_shape=[jax.ShapeDtypeStruct((_B, _S, 1), jnp.float32)
                   for _ in range(2)],
    )(y, y_prev, linguistic_anchor)
    dn = dn3.reshape(_B, _S)
    dr = dr3.reshape(_B, _S)
    iidx = jnp.asarray(iter_idx, jnp.int32).reshape(1, 1)
    out = pl.pallas_call(
        _route_body,
        in_specs=[
            pl.BlockSpec((_B, _S), lambda: (0, 0)),
            pl.BlockSpec((_B, _S), lambda: (0, 0)),
            pl.BlockSpec((_MAXIT, 16), lambda: (0, 0)),
            pl.BlockSpec((64, 18), lambda: (0, 0)),
            pl.BlockSpec((1, 64), lambda: (0, 0)),
            pl.BlockSpec((_NE, 32), lambda: (0, 0)),
            pl.BlockSpec((1, 1), lambda: (0, 0)),
        ],
        out_specs=pl.BlockSpec((8, 128), lambda: (0, 0)),
        out_shape=jax.ShapeDtypeStruct((8, 128), jnp.float32),
    )(dn, dr, iter_table, W1, b1.reshape(1, 64), W2, iidx)
    return out[0:_B, 0:_NE]


# speedup vs baseline: 1.1248x; 1.1248x over previous
import jax
import jax.numpy as jnp
from jax.experimental import pallas as pl
from jax.experimental.pallas import tpu as pltpu

_DIM = 2048
_NE = 8
_MAXIT = 3
_B = 4
_S = 4096
_SBLK = 512
_NS = _S // _SBLK


def _norms_body(y_ref, yp_ref, an_ref, pd_ref, pa_ref):
    y = y_ref[0]
    d = y - yp_ref[0]
    a = y - an_ref[0]
    dn = jnp.sqrt(jnp.sum(d * d, axis=1, keepdims=True))
    an = jnp.sqrt(jnp.sum(a * a, axis=1, keepdims=True))
    pd_ref[0, 0, 0, 0] = jnp.sum(dn)
    pa_ref[0, 0, 0, 0] = jnp.sum(an)


def _route_body(pd_ref, pa_ref, it_ref, w1_ref, b1_ref, w2_ref, idx_ref, out_ref):
    delta = jnp.sum(pd_ref[...], axis=1, keepdims=True) * (1.0 / _S)
    drift = jnp.sum(pa_ref[...], axis=1, keepdims=True) * (1.0 / _S)
    delta = delta / (jnp.mean(delta) + 1e-8)
    drift = drift / (jnp.mean(drift) + 1e-8)
    clamped = jnp.minimum(idx_ref[...], _MAXIT - 1)
    sel = (jax.lax.broadcasted_iota(jnp.int32, (1, _MAXIT), 1) == clamped
           ).astype(jnp.float32)
    emb = jax.lax.dot_general(sel, it_ref[...], (((1,), (0,)), ((), ())),
                              preferred_element_type=jnp.float32)
    emb4 = jnp.broadcast_to(emb, (_B, 16))
    x = jnp.concatenate([delta, drift, emb4], axis=1)
    h = jax.lax.dot_general(x, w1_ref[...], (((1,), (1,)), ((), ())),
                            preferred_element_type=jnp.float32) + b1_ref[...]
    xh = h[:, : _NE * 4]
    gate = h[:, _NE * 4:]
    h2 = (gate * jax.lax.logistic(gate)) * xh
    logits = jax.lax.dot_general(h2, w2_ref[...], (((1,), (1,)), ((), ())),
                                 preferred_element_type=jnp.float32)
    mx = jnp.max(logits, axis=1, keepdims=True)
    iota = jax.lax.broadcasted_iota(jnp.int32, (_B, _NE), 1)
    first = jnp.min(jnp.where(logits == mx, iota, _NE), axis=1, keepdims=True)
    onehot = (iota == first).astype(jnp.float32)
    out_ref[...] = jnp.zeros((8, 128), jnp.float32)
    out_ref[0:_B, 0:_NE] = onehot


def kernel(y, y_prev, linguistic_anchor, iter_table, W1, b1, W2, iter_idx):
    pd, pa = pl.pallas_call(
        _norms_body,
        grid=(_B, _NS),
        in_specs=[pl.BlockSpec((1, _SBLK, _DIM), lambda b, s: (b, s, 0))
                  for _ in range(3)],
        out_specs=[pl.BlockSpec((1, 1, 1, 1), lambda b, s: (b, s, 0, 0),
                                memory_space=pltpu.SMEM)
                   for _ in range(2)],
        out_shape=[jax.ShapeDtypeStruct((_B, _NS, 1, 1), jnp.float32)
                   for _ in range(2)],
    )(y, y_prev, linguistic_anchor)
    pd = pd.reshape(_B, _NS)
    pa = pa.reshape(_B, _NS)
    iidx = jnp.asarray(iter_idx, jnp.int32).reshape(1, 1)
    out = pl.pallas_call(
        _route_body,
        in_specs=[
            pl.BlockSpec((_B, _NS), lambda: (0, 0)),
            pl.BlockSpec((_B, _NS), lambda: (0, 0)),
            pl.BlockSpec((_MAXIT, 16), lambda: (0, 0)),
            pl.BlockSpec((64, 18), lambda: (0, 0)),
            pl.BlockSpec((1, 64), lambda: (0, 0)),
            pl.BlockSpec((_NE, 32), lambda: (0, 0)),
            pl.BlockSpec((1, 1), lambda: (0, 0)),
        ],
        out_specs=pl.BlockSpec((8, 128), lambda: (0, 0)),
        out_shape=jax.ShapeDtypeStruct((8, 128), jnp.float32),
    )(pd, pa, iter_table, W1, b1.reshape(1, 64), W2, iidx)
    return out[0:_B, 0:_NE]
